# baseline (speedup 1.0000x reference)
import functools

import jax
import jax.numpy as jnp
from jax import lax
from jax.experimental import pallas as pl
from jax.experimental.pallas import tpu as pltpu
from jax.experimental.pallas import tpu_sc as plsc

_B, _N, _M = 4, 4096, 4096
_NW = 32
_CH = _N // (_NW // _B)
_NCK = _B * _N // _CH


def _rne_bf16(v):
    bits = lax.bitcast_convert_type(v, jnp.int32)
    lsb = lax.shift_right_logical(bits, 16) & 1
    rounded = (bits + 0x7FFF + lsb) & jnp.int32(-65536)
    return lax.bitcast_convert_type(rounded, jnp.float32)


def _round_chunk(src_ref, norm_ref, n):
    def body(i, _):
        sl = pl.ds(i * 16, 16)
        v = src_ref[sl]
        norm_ref[sl] = norm_ref[sl] + v * v
        src_ref[sl] = _rne_bf16(v * -2.0)
        return 0
    lax.fori_loop(0, n // 16, body, 0)


def _round_full(src_ref, norm_ref, n):
    def body(i, _):
        sl = pl.ds(i * 16, 16)
        v = src_ref[sl]
        norm_ref[sl] = norm_ref[sl] + v * v
        src_ref[sl] = _rne_bf16(v)
        return 0
    lax.fori_loop(0, n // 16, body, 0)


def _zero(ref, n):
    def body(i, _):
        ref[pl.ds(i * 16, 16)] = jnp.zeros((16,), jnp.float32)
        return 0
    lax.fori_loop(0, n // 16, body, 0)


def _make_sc_kernel():
    mesh = plsc.VectorSubcoreMesh(core_axis_name="c", subcore_axis_name="s",
                                  num_cores=2)

    @functools.partial(
        pl.kernel, mesh=mesh,
        out_type=jax.ShapeDtypeStruct((_NW, 16), jnp.float32),
        scratch_types=[
            pltpu.VMEM((_CH,), jnp.float32),
            pltpu.VMEM((_CH,), jnp.float32),
            pltpu.VMEM((_CH,), jnp.float32),
            pltpu.VMEM((_CH,), jnp.float32),
            pltpu.VMEM((_M,), jnp.float32),
            pltpu.VMEM((_M,), jnp.float32),
            pltpu.VMEM((_M,), jnp.float32),
            pltpu.VMEM((_M,), jnp.float32),
            pltpu.VMEM((16,), jnp.float32),
        ],
    )
    def sc_kernel(xf_hbm, yf_hbm, out_hbm,
                  c0, c1, c2, cn, f0, f1, f2, fn, ps):
        wid = lax.axis_index("s") * 2 + lax.axis_index("c")
        b = wid // 8
        slot = wid % 8

        psum = jnp.zeros((16,), jnp.float32)

        for direction in (0, 1):
            chunk_hbm = xf_hbm if direction == 0 else yf_hbm
            full_hbm = yf_hbm if direction == 0 else xf_hbm
            nfull = _M if direction == 0 else _N
            inv = 1.0 / (_N * _B) if direction == 0 else 1.0 / (_M * _B)

            cbase = b * 3 * (_N if direction == 0 else _M) + slot * _CH
            fbase = b * 3 * nfull
            nd = _N if direction == 0 else _M

            pltpu.sync_copy(chunk_hbm.at[pl.ds(cbase, _CH)], c0)
            pltpu.sync_copy(chunk_hbm.at[pl.ds(cbase + nd, _CH)], c1)
            pltpu.sync_copy(chunk_hbm.at[pl.ds(cbase + 2 * nd, _CH)], c2)
            pltpu.sync_copy(full_hbm.at[pl.ds(fbase, nfull)], f0)
            pltpu.sync_copy(full_hbm.at[pl.ds(fbase + nfull, nfull)], f1)
            pltpu.sync_copy(full_hbm.at[pl.ds(fbase + 2 * nfull, nfull)], f2)

            _zero(cn, _CH)
            _zero(fn, nfull)
            _round_chunk(c0, cn, _CH)
            _round_chunk(c1, cn, _CH)
            _round_chunk(c2, cn, _CH)
            _round_full(f0, fn, nfull)
            _round_full(f1, fn, nfull)
            _round_full(f2, fn, nfull)

            def group_body(g, acc_psum):
                gsl = pl.ds(g * 16, 16)
                cm0v = c0[gsl]
                cm1v = c1[gsl]
                cm2v = c2[gsl]
                cnv = cn[gsl]

                def scan_body(jv, acc):
                    sl = pl.ds(jv * 16, 16)
                    g0 = f0[sl]
                    g1 = f1[sl]
                    g2 = f2[sl]
                    gn = fn[sl]
                    ds_ = []
                    for l in range(16):
                        xy = (cm0v * g0[l] + cm1v * g1[l]) + cm2v * g2[l]
                        ds_.append((cnv + gn[l]) + xy)
                    while len(ds_) > 1:
                        ds_ = [jnp.minimum(ds_[k], ds_[k + 1])
                               for k in range(0, len(ds_), 2)]
                    return jnp.minimum(acc, ds_[0])

                acc0 = jnp.full((16,), jnp.inf, jnp.float32)
                acc = lax.fori_loop(0, nfull // 16, scan_body, acc0)
                acc_psum = acc_psum + jnp.maximum(acc, 0.0) * inv
                return acc_psum

            psum = lax.fori_loop(0, _CH // 16, group_body, psum)

        ps[...] = psum
        pltpu.sync_copy(ps, out_hbm.at[wid])

    return sc_kernel


def kernel(x, y):
    B, N, _ = x.shape
    M = y.shape[1]
    xf = jnp.transpose(x, (0, 2, 1)).reshape(-1)
    yf = jnp.transpose(y, (0, 2, 1)).reshape(-1)
    out = _make_sc_kernel()(xf, yf)
    return jnp.sum(out)


# speedup vs baseline: 17.8667x; 17.8667x over previous
import functools

import jax
import jax.numpy as jnp
from jax.experimental import pallas as pl
from jax.experimental.pallas import tpu as pltpu

_TI = 2048
_TC = 1024


def _chamfer_body(x_ref, yt_ref, out_ref, colmin_ref, *,
                  ni, m, inv_xn, inv_ym):
    b = pl.program_id(0)
    i = pl.program_id(1)

    xs = x_ref[0]
    ys = yt_ref[0]

    x2 = jnp.sum(xs * xs, axis=1, keepdims=True)
    y2 = jnp.sum(ys * ys, axis=0, keepdims=True)
    x2h = x2.astype(jnp.bfloat16)
    x2l = (x2 - x2h.astype(jnp.float32)).astype(jnp.bfloat16)
    y2h = y2.astype(jnp.bfloat16)
    y2l = (y2 - y2h.astype(jnp.float32)).astype(jnp.bfloat16)
    ones_x = jnp.ones(x2h.shape, jnp.bfloat16)
    a = jnp.concatenate(
        [(xs * -2.0).astype(jnp.bfloat16), x2h, x2l, ones_x, ones_x],
        axis=1)
    ones_y = jnp.ones(y2h.shape, jnp.bfloat16)
    bmat = jnp.concatenate(
        [ys.astype(jnp.bfloat16), ones_y, ones_y, y2h, y2l],
        axis=0)

    rowmin = None
    cols = []
    for c in range(m // _TC):
        dc = jax.lax.dot_general(
            a, bmat[:, c * _TC:(c + 1) * _TC],
            dimension_numbers=(((1,), (0,)), ((), ())),
            preferred_element_type=jnp.float32)
        rc = jnp.min(dc, axis=1, keepdims=True)
        rowmin = rc if rowmin is None else jnp.minimum(rowmin, rc)
        cols.append(jnp.min(dc, axis=0, keepdims=True))
    col = jnp.concatenate(cols, axis=1)

    rowmin = jnp.maximum(rowmin, 0.0)

    @pl.when(jnp.logical_and(b == 0, i == 0))
    def _():
        out_ref[0, 0] = 0.0

    out_ref[0, 0] += jnp.sum(rowmin) * inv_xn

    @pl.when(i == 0)
    def _():
        colmin_ref[...] = col

    @pl.when(i > 0)
    def _():
        colmin_ref[...] = jnp.minimum(colmin_ref[...], col)

    @pl.when(i == ni - 1)
    def _():
        out_ref[0, 0] += jnp.sum(jnp.maximum(colmin_ref[...], 0.0)) * inv_ym


def kernel(x, y):
    B, N, _ = x.shape
    M = y.shape[1]
    ni = N // _TI
    yt = jnp.transpose(y, (0, 2, 1))

    out = pl.pallas_call(
        functools.partial(_chamfer_body, ni=ni, m=M,
                          inv_xn=1.0 / (N * B), inv_ym=1.0 / (M * B)),
        grid=(B, ni),
        in_specs=[
            pl.BlockSpec((1, _TI, 3), lambda b, i: (b, i, 0)),
            pl.BlockSpec((1, 3, M), lambda b, i: (b, 0, 0)),
        ],
        out_specs=pl.BlockSpec((1, 1), lambda b, i: (0, 0),
                               memory_space=pltpu.SMEM),
        out_shape=jax.ShapeDtypeStruct((1, 1), jnp.float32),
        scratch_shapes=[
            pltpu.VMEM((1, M), jnp.float32),
        ],
    )(x, yt)
    return out[0, 0]


# speedup vs baseline: 18.6954x; 1.0464x over previous
import functools

import jax
import jax.numpy as jnp
from jax.experimental import pallas as pl
from jax.experimental.pallas import tpu as pltpu

_TI = 4096
_TC = 1024


def _chamfer_body(x_ref, yt_ref, out_ref, colmin_ref, *,
                  ni, m, inv_xn, inv_ym):
    b = pl.program_id(0)
    i = pl.program_id(1)

    xs = x_ref[0]
    ys = yt_ref[0]

    x2 = jnp.sum(xs * xs, axis=1, keepdims=True)
    y2 = jnp.sum(ys * ys, axis=0, keepdims=True)
    x2h = x2.astype(jnp.bfloat16)
    x2l = (x2 - x2h.astype(jnp.float32)).astype(jnp.bfloat16)
    y2h = y2.astype(jnp.bfloat16)
    y2l = (y2 - y2h.astype(jnp.float32)).astype(jnp.bfloat16)
    ones_x = jnp.ones(x2h.shape, jnp.bfloat16)
    a = jnp.concatenate(
        [(xs * -2.0).astype(jnp.bfloat16), x2h, x2l, ones_x, ones_x],
        axis=1)
    ones_y = jnp.ones(y2h.shape, jnp.bfloat16)
    bmat = jnp.concatenate(
        [ys.astype(jnp.bfloat16), ones_y, ones_y, y2h, y2l],
        axis=0)

    rowmin = None
    cols = []
    for c in range(m // _TC):
        dc = jax.lax.dot_general(
            a, bmat[:, c * _TC:(c + 1) * _TC],
            dimension_numbers=(((1,), (0,)), ((), ())),
            preferred_element_type=jnp.float32)
        rc = jnp.min(dc, axis=1, keepdims=True)
        rowmin = rc if rowmin is None else jnp.minimum(rowmin, rc)
        cols.append(jnp.min(dc, axis=0, keepdims=True))
    col = jnp.concatenate(cols, axis=1)

    rowmin = jnp.maximum(rowmin, 0.0)

    @pl.when(jnp.logical_and(b == 0, i == 0))
    def _():
        out_ref[0, 0] = 0.0

    out_ref[0, 0] += jnp.sum(rowmin) * inv_xn

    @pl.when(i == 0)
    def _():
        colmin_ref[...] = col

    @pl.when(i > 0)
    def _():
        colmin_ref[...] = jnp.minimum(colmin_ref[...], col)

    @pl.when(i == ni - 1)
    def _():
        out_ref[0, 0] += jnp.sum(jnp.maximum(colmin_ref[...], 0.0)) * inv_ym


def kernel(x, y):
    B, N, _ = x.shape
    M = y.shape[1]
    ni = N // _TI
    yt = jnp.transpose(y, (0, 2, 1))

    out = pl.pallas_call(
        functools.partial(_chamfer_body, ni=ni, m=M,
                          inv_xn=1.0 / (N * B), inv_ym=1.0 / (M * B)),
        grid=(B, ni),
        in_specs=[
            pl.BlockSpec((1, _TI, 3), lambda b, i: (b, i, 0)),
            pl.BlockSpec((1, 3, M), lambda b, i: (b, 0, 0)),
        ],
        out_specs=pl.BlockSpec((1, 1), lambda b, i: (0, 0),
                               memory_space=pltpu.SMEM),
        out_shape=jax.ShapeDtypeStruct((1, 1), jnp.float32),
        scratch_shapes=[
            pltpu.VMEM((1, M), jnp.float32),
        ],
    )(x, yt)
    return out[0, 0]
